# baseline (speedup 1.0000x reference)
import functools
import math

import jax
import jax.numpy as jnp
from jax import lax
from jax.experimental import pallas as pl
from jax.experimental.pallas import tpu as pltpu
from jax.experimental.pallas import tpu_sc as plsc

N_NODES = 10000
N_EDGES = 320000
D_FEAT = 128
D_ATTR = 16
INV_SQRT_AVG = 1.0 / math.sqrt(32.0)

NC = 2
NS = 16
B_IDX = 128
NRING = 3
N_BLOCKS = N_EDGES // B_IDX
HI_TILES = 4
ROWS_PER_TILE = 624
TAIL_ROWS = N_NODES - NS * ROWS_PER_TILE
ZROWS = 208


def _sc_segment_sum(edge_message, edge_dst):
    mesh = plsc.VectorSubcoreMesh(
        core_axis_name="c", subcore_axis_name="s", num_cores=NC,
        num_subcores=NS)

    @functools.partial(
        pl.kernel,
        out_type=jax.ShapeDtypeStruct((NC * N_NODES, D_FEAT), jnp.float32),
        mesh=mesh,
        scratch_types=[
            pltpu.VMEM_SHARED((N_NODES, D_FEAT), jnp.float32),
            pltpu.VMEM((NRING, B_IDX, D_FEAT), jnp.float32),
            pltpu.VMEM((NRING, 2, B_IDX), jnp.int32),
            pltpu.SemaphoreType.DMA((NRING,)),
            pltpu.SemaphoreType.DMA((NRING,)),
            pltpu.SemaphoreType.DMA((NRING,)),
        ],
    )
    def k(msg_hbm, idx_hbm, out_hbm, acc, rows_v, idxg,
          ldi_sem, ldr_sem, scat_sem):
        c = lax.axis_index("c")
        s = lax.axis_index("s")
        wid = c * NS + s
        blk_start = 78 * wid + jnp.minimum(wid, HI_TILES)
        n_blk = jnp.where(wid < HI_TILES, 79, 78)

        def slot(i):
            return lax.rem(i, NRING)

        def issue_load(i, b):
            base = (blk_start + i) * B_IDX
            pltpu.async_copy(idx_hbm.at[pl.ds(0, 2), pl.ds(base, B_IDX)],
                             idxg.at[b], ldi_sem.at[b])
            pltpu.async_copy(msg_hbm.at[pl.ds(base, B_IDX)], rows_v.at[b],
                             ldr_sem.at[b])

        def wait_load(i, b):
            base = (blk_start + i) * B_IDX
            pltpu.make_async_copy(
                idx_hbm.at[pl.ds(0, 2), pl.ds(base, B_IDX)],
                idxg.at[b], ldi_sem.at[b]).wait()
            pltpu.make_async_copy(msg_hbm.at[pl.ds(base, B_IDX)],
                                  rows_v.at[b], ldr_sem.at[b]).wait()

        def issue_scat(b):
            pltpu.async_copy(rows_v.at[b], acc.at[idxg.at[b, 1]],
                             scat_sem.at[b], add=True)

        def wait_scat(b):
            pltpu.make_async_copy(rows_v.at[b], acc.at[idxg.at[b, 1]],
                                  scat_sem.at[b]).wait()

        for i in range(NRING - 1):
            issue_load(i, i)

        def zero_store(i, _):
            r = i // 8
            col = (i % 8) * 16
            rows_v[NRING - 1, r, pl.ds(col, 16)] = jnp.zeros(
                (16,), jnp.float32)
            return 0
        lax.fori_loop(0, B_IDX * 8, zero_store, 0)
        zsrc = rows_v.at[NRING - 1]
        for z in range(ROWS_PER_TILE // B_IDX):
            pltpu.sync_copy(
                zsrc, acc.at[pl.ds(s * ROWS_PER_TILE + z * B_IDX, B_IDX)])
        rem = ROWS_PER_TILE - (ROWS_PER_TILE // B_IDX) * B_IDX
        pltpu.sync_copy(
            zsrc.at[pl.ds(0, rem)],
            acc.at[pl.ds(s * ROWS_PER_TILE + ROWS_PER_TILE - rem, rem)])

        @pl.when(s == NS - 1)
        def _zero_tail():
            pltpu.sync_copy(zsrc.at[pl.ds(0, TAIL_ROWS)],
                            acc.at[pl.ds(NS * ROWS_PER_TILE, TAIL_ROWS)])
        plsc.subcore_barrier()

        def block_body(i, _):
            b = slot(i)
            wait_load(i, b)
            issue_scat(b)
            nxt = i + NRING - 1
            bn = slot(nxt)

            @pl.when(nxt < n_blk)
            def _issue_next():
                @pl.when(i >= 1)
                def _wait_prev_scat():
                    wait_scat(bn)
                issue_load(nxt, bn)
            return 0
        lax.fori_loop(0, n_blk, block_body, 0)
        for b in range(NRING):
            wait_scat(b)
        plsc.subcore_barrier()

        out_base = c * N_NODES + s * ROWS_PER_TILE
        pltpu.sync_copy(acc.at[pl.ds(s * ROWS_PER_TILE, ROWS_PER_TILE)],
                        out_hbm.at[pl.ds(out_base, ROWS_PER_TILE)])

        @pl.when(s == NS - 1)
        def _drain_tail():
            pltpu.sync_copy(
                acc.at[pl.ds(NS * ROWS_PER_TILE, TAIL_ROWS)],
                out_hbm.at[pl.ds(c * N_NODES + NS * ROWS_PER_TILE,
                                 TAIL_ROWS)])

    return k(edge_message, edge_dst)


def _tc_update(partials, node_attrs, W_tp, W_lin):
    R = 1000
    nblk = N_NODES // R

    def body(p0_ref, p1_ref, attrs_ref, wtp_ref, wlin_ref, out_ref):
        m = (p0_ref[...] + p1_ref[...]) * INV_SQRT_AVG
        a = lax.dot_general(attrs_ref[...], wtp_ref[...],
                            (((1,), (1,)), ((), ())),
                            preferred_element_type=jnp.float32)
        f = m * a
        f = lax.dot_general(f, wlin_ref[...], (((1,), (0,)), ((), ())),
                            preferred_element_type=jnp.float32)
        out_ref[...] = f * jax.nn.sigmoid(f)

    return pl.pallas_call(
        body,
        grid=(nblk,),
        in_specs=[
            pl.BlockSpec((R, D_FEAT), lambda i: (i, 0)),
            pl.BlockSpec((R, D_FEAT), lambda i: (i + nblk, 0)),
            pl.BlockSpec((R, D_ATTR), lambda i: (i, 0)),
            pl.BlockSpec((D_FEAT, D_ATTR), lambda i: (0, 0)),
            pl.BlockSpec((D_FEAT, D_FEAT), lambda i: (0, 0)),
        ],
        out_specs=pl.BlockSpec((R, D_FEAT), lambda i: (i, 0)),
        out_shape=jax.ShapeDtypeStruct((N_NODES, D_FEAT), jnp.float32),
    )(partials, partials, node_attrs, W_tp, W_lin)


def kernel(node_feats, node_attrs, edge_message, edge_index, W_tp, W_lin):
    del node_feats
    partials = _sc_segment_sum(edge_message, edge_index)
    return _tc_update(partials, node_attrs, W_tp, W_lin)


# speedup vs baseline: 1.1239x; 1.1239x over previous
import functools
import math

import jax
import jax.numpy as jnp
from jax import lax
from jax.experimental import pallas as pl
from jax.experimental.pallas import tpu as pltpu
from jax.experimental.pallas import tpu_sc as plsc

N_NODES = 10000
N_EDGES = 320000
D_FEAT = 128
D_ATTR = 16
INV_SQRT_AVG = 1.0 / math.sqrt(32.0)

NC = 2
NS = 16
B_IDX = 128
NRING = 3
N_BLOCKS = N_EDGES // B_IDX
HI_TILES = 4
ROWS_PER_TILE = 624
TAIL_ROWS = N_NODES - NS * ROWS_PER_TILE
ZROWS = 208


def _sc_segment_sum(edge_message, edge_dst):
    mesh = plsc.VectorSubcoreMesh(
        core_axis_name="c", subcore_axis_name="s", num_cores=NC,
        num_subcores=NS)

    @functools.partial(
        pl.kernel,
        out_type=jax.ShapeDtypeStruct((NC * N_NODES, D_FEAT), jnp.float32),
        mesh=mesh,
        scratch_types=[
            pltpu.VMEM_SHARED((N_NODES, D_FEAT), jnp.float32),
            pltpu.VMEM((NRING, B_IDX, D_FEAT), jnp.float32),
            pltpu.VMEM((NRING, 2, B_IDX), jnp.int32),
            pltpu.SemaphoreType.DMA((NRING,)),
            pltpu.SemaphoreType.DMA((NRING,)),
            pltpu.SemaphoreType.DMA((NRING,)),
        ],
    )
    def k(msg_hbm, idx_hbm, out_hbm, acc, rows_v, idxg,
          ldi_sem, ldr_sem, scat_sem):
        c = lax.axis_index("c")
        s = lax.axis_index("s")
        wid = c * NS + s
        blk_start = 78 * wid + jnp.minimum(wid, HI_TILES)
        n_blk = jnp.where(wid < HI_TILES, 79, 78)

        def slot(i):
            return lax.rem(i, NRING)

        def issue_load(i, b):
            base = (blk_start + i) * B_IDX
            pltpu.async_copy(idx_hbm.at[pl.ds(0, 2), pl.ds(base, B_IDX)],
                             idxg.at[b], ldi_sem.at[b])
            pltpu.async_copy(msg_hbm.at[pl.ds(base, B_IDX)], rows_v.at[b],
                             ldr_sem.at[b])

        def wait_load(i, b):
            base = (blk_start + i) * B_IDX
            pltpu.make_async_copy(
                idx_hbm.at[pl.ds(0, 2), pl.ds(base, B_IDX)],
                idxg.at[b], ldi_sem.at[b]).wait()
            pltpu.make_async_copy(msg_hbm.at[pl.ds(base, B_IDX)],
                                  rows_v.at[b], ldr_sem.at[b]).wait()

        def issue_scat(b):
            pltpu.async_copy(rows_v.at[b], acc.at[idxg.at[b, 1]],
                             scat_sem.at[b], add=True)

        def wait_scat(b):
            pltpu.make_async_copy(rows_v.at[b], acc.at[idxg.at[b, 1]],
                                  scat_sem.at[b]).wait()

        for i in range(NRING - 1):
            issue_load(i, i)

        def zero_store(i, _):
            r = i // 8
            col = (i % 8) * 16
            rows_v[NRING - 1, r, pl.ds(col, 16)] = jnp.zeros(
                (16,), jnp.float32)
            return 0
        lax.fori_loop(0, B_IDX * 8, zero_store, 0)
        zsrc = rows_v.at[NRING - 1]
        for z in range(ROWS_PER_TILE // B_IDX):
            pltpu.sync_copy(
                zsrc, acc.at[pl.ds(s * ROWS_PER_TILE + z * B_IDX, B_IDX)])
        rem = ROWS_PER_TILE - (ROWS_PER_TILE // B_IDX) * B_IDX
        pltpu.sync_copy(
            zsrc.at[pl.ds(0, rem)],
            acc.at[pl.ds(s * ROWS_PER_TILE + ROWS_PER_TILE - rem, rem)])

        @pl.when(s == NS - 1)
        def _zero_tail():
            pltpu.sync_copy(zsrc.at[pl.ds(0, TAIL_ROWS)],
                            acc.at[pl.ds(NS * ROWS_PER_TILE, TAIL_ROWS)])
        plsc.subcore_barrier()

        def block_body(i, _):
            b = slot(i)
            nxt = i + NRING - 1
            bn = slot(nxt)

            @pl.when(nxt < n_blk)
            def _issue_next():
                @pl.when(i >= 1)
                def _wait_prev_scat():
                    wait_scat(bn)
                issue_load(nxt, bn)
            wait_load(i, b)
            issue_scat(b)
            return 0
        lax.fori_loop(0, n_blk, block_body, 0)
        for b in range(NRING):
            wait_scat(b)
        plsc.subcore_barrier()

        out_base = c * N_NODES + s * ROWS_PER_TILE
        pltpu.sync_copy(acc.at[pl.ds(s * ROWS_PER_TILE, ROWS_PER_TILE)],
                        out_hbm.at[pl.ds(out_base, ROWS_PER_TILE)])

        @pl.when(s == NS - 1)
        def _drain_tail():
            pltpu.sync_copy(
                acc.at[pl.ds(NS * ROWS_PER_TILE, TAIL_ROWS)],
                out_hbm.at[pl.ds(c * N_NODES + NS * ROWS_PER_TILE,
                                 TAIL_ROWS)])

    return k(edge_message, edge_dst)


def _tc_update(partials, node_attrs, W_tp, W_lin):
    R = 2000
    nblk = N_NODES // R

    def body(p0_ref, p1_ref, attrs_ref, wtp_ref, wlin_ref, out_ref):
        m = (p0_ref[...] + p1_ref[...]) * INV_SQRT_AVG
        a = lax.dot_general(attrs_ref[...], wtp_ref[...],
                            (((1,), (1,)), ((), ())),
                            preferred_element_type=jnp.float32)
        f = m * a
        f = lax.dot_general(f, wlin_ref[...], (((1,), (0,)), ((), ())),
                            preferred_element_type=jnp.float32)
        out_ref[...] = f * jax.nn.sigmoid(f)

    return pl.pallas_call(
        body,
        grid=(nblk,),
        in_specs=[
            pl.BlockSpec((R, D_FEAT), lambda i: (i, 0)),
            pl.BlockSpec((R, D_FEAT), lambda i: (i + nblk, 0)),
            pl.BlockSpec((R, D_ATTR), lambda i: (i, 0)),
            pl.BlockSpec((D_FEAT, D_ATTR), lambda i: (0, 0)),
            pl.BlockSpec((D_FEAT, D_FEAT), lambda i: (0, 0)),
        ],
        out_specs=pl.BlockSpec((R, D_FEAT), lambda i: (i, 0)),
        out_shape=jax.ShapeDtypeStruct((N_NODES, D_FEAT), jnp.float32),
    )(partials, partials, node_attrs, W_tp, W_lin)


def kernel(node_feats, node_attrs, edge_message, edge_index, W_tp, W_lin):
    del node_feats
    partials = _sc_segment_sum(edge_message, edge_index)
    return _tc_update(partials, node_attrs, W_tp, W_lin)
